# speedup vs baseline: 19.2522x; 19.2522x over previous
import functools

import jax
import jax.numpy as jnp
from jax.experimental import pallas as pl
from jax.experimental.pallas import tpu as pltpu

_RS = 128


def _krylov_chain_kernel(n_strips, n_blocks, in_f,
                         adj_ref, x_ref, w_ref, bias_ref, out_ref,
                         a_bf, x_bf, ycat):
    s = pl.program_id(0)

    @pl.when(s == 0)
    def _():
        x_bf[...] = x_ref[...].astype(jnp.bfloat16)

    @pl.when(s < n_strips)
    def _():
        blk = adj_ref[...].astype(jnp.bfloat16)
        a_bf[pl.ds(s * _RS, _RS), :] = blk
        y1 = jnp.dot(blk, x_bf[...], preferred_element_type=jnp.float32)
        ycat[pl.ds(s * _RS, _RS), 0:in_f] = y1.astype(jnp.bfloat16)

    for b in range(1, n_blocks):
        @pl.when(s == n_strips + b - 1)
        def _(b=b):
            y = jnp.dot(a_bf[...], ycat[:, (b - 1) * in_f:b * in_f],
                        preferred_element_type=jnp.float32)
            ycat[:, b * in_f:(b + 1) * in_f] = y.astype(jnp.bfloat16)

    @pl.when(s == n_strips + n_blocks - 2)
    def _():
        w_bf = w_ref[...].astype(jnp.bfloat16)
        out_ref[...] = (jnp.dot(ycat[...], w_bf,
                                preferred_element_type=jnp.float32)
                        + bias_ref[...])


def kernel(x, adj, shared_weight, output_bias, a_exp):
    n, in_f = x.shape
    out_f = shared_weight.shape[1]
    n_blocks = shared_weight.shape[0] // in_f
    n_strips = n // _RS

    bias_row = output_bias.reshape(1, out_f).astype(jnp.float32)
    body = functools.partial(_krylov_chain_kernel, n_strips, n_blocks, in_f)

    return pl.pallas_call(
        body,
        out_shape=jax.ShapeDtypeStruct((n, out_f), jnp.float32),
        grid_spec=pltpu.PrefetchScalarGridSpec(
            num_scalar_prefetch=0,
            grid=(n_strips + n_blocks - 1,),
            in_specs=[
                pl.BlockSpec((_RS, n),
                             lambda s: (jnp.minimum(s, n_strips - 1), 0)),
                pl.BlockSpec((n, in_f), lambda s: (0, 0)),
                pl.BlockSpec(shared_weight.shape, lambda s: (0, 0)),
                pl.BlockSpec((1, out_f), lambda s: (0, 0)),
            ],
            out_specs=pl.BlockSpec((n, out_f), lambda s: (0, 0)),
            scratch_shapes=[
                pltpu.VMEM((n, n), jnp.bfloat16),
                pltpu.VMEM((n, in_f), jnp.bfloat16),
                pltpu.VMEM((n, n_blocks * in_f), jnp.bfloat16),
            ],
        ),
        compiler_params=pltpu.CompilerParams(
            dimension_semantics=("arbitrary",)),
    )(adj, x.astype(jnp.float32), shared_weight.astype(jnp.float32), bias_row)


# speedup vs baseline: 20.3419x; 1.0566x over previous
import functools

import jax
import jax.numpy as jnp
from jax.experimental import pallas as pl
from jax.experimental.pallas import tpu as pltpu

_RS = 256


def _krylov_body(n, n_strips, n_blocks, in_f,
                 adj_hbm, x_ref, w_ref, bias_ref, out_ref,
                 a_bf, abuf, x_bf, ycat, sem):
    def start_strip(slot, step):
        pltpu.make_async_copy(adj_hbm.at[pl.ds(step * _RS, _RS), :],
                              abuf.at[slot], sem.at[slot]).start()

    def wait_strip(slot):
        pltpu.make_async_copy(adj_hbm.at[pl.ds(0, _RS), :],
                              abuf.at[slot], sem.at[slot]).wait()

    start_strip(0, 0)
    x_bf[...] = x_ref[...].astype(jnp.bfloat16)

    def strip_step(step, carry):
        cur = jax.lax.rem(step, 2)
        nxt = jax.lax.rem(step + 1, 2)

        @pl.when(step + 1 < n_strips)
        def _():
            start_strip(nxt, step + 1)

        wait_strip(cur)
        blk = abuf[cur].astype(jnp.bfloat16)
        a_bf[pl.ds(step * _RS, _RS), :] = blk
        y1 = jnp.dot(blk, x_bf[...], preferred_element_type=jnp.float32)
        ycat[pl.ds(step * _RS, _RS), 0:in_f] = y1.astype(jnp.bfloat16)
        return carry

    jax.lax.fori_loop(0, n_strips, strip_step, 0)

    for b in range(1, n_blocks):
        y = jnp.dot(a_bf[...], ycat[:, (b - 1) * in_f:b * in_f],
                    preferred_element_type=jnp.float32)
        ycat[:, b * in_f:(b + 1) * in_f] = y.astype(jnp.bfloat16)

    out_ref[...] = (jnp.dot(ycat[...], w_ref[...].astype(jnp.bfloat16),
                            preferred_element_type=jnp.float32)
                    + bias_ref[...])


def kernel(x, adj, shared_weight, output_bias, a_exp):
    n, in_f = x.shape
    out_f = shared_weight.shape[1]
    n_blocks = shared_weight.shape[0] // in_f
    n_strips = n // _RS

    bias_row = output_bias.reshape(1, out_f).astype(jnp.float32)
    body = functools.partial(_krylov_body, n, n_strips, n_blocks, in_f)

    return pl.pallas_call(
        body,
        out_shape=jax.ShapeDtypeStruct((n, out_f), jnp.float32),
        in_specs=[
            pl.BlockSpec(memory_space=pltpu.MemorySpace.HBM),
            pl.BlockSpec((n, in_f), lambda: (0, 0)),
            pl.BlockSpec(shared_weight.shape, lambda: (0, 0)),
            pl.BlockSpec((1, out_f), lambda: (0, 0)),
        ],
        out_specs=pl.BlockSpec((n, out_f), lambda: (0, 0)),
        scratch_shapes=[
            pltpu.VMEM((n, n), jnp.bfloat16),
            pltpu.VMEM((2, _RS, n), jnp.float32),
            pltpu.VMEM((n, in_f), jnp.bfloat16),
            pltpu.VMEM((n, n_blocks * in_f), jnp.bfloat16),
            pltpu.SemaphoreType.DMA((2,)),
        ],
    )(adj, x.astype(jnp.float32), shared_weight.astype(jnp.float32), bias_row)
